# baseline (speedup 1.0000x reference)
import functools

import jax
import jax.numpy as jnp
from jax import lax
from jax.experimental import pallas as pl
from jax.experimental.pallas import tpu as pltpu
from jax.experimental.pallas import tpu_sc as plsc

MI_N = 2000
CI_N = 3000
MC_N = MI_N + CI_N

NC = 2
NS = 16
NW = NC * NS
CH = 128

BN = 256
EC = 256


def _round_up(x, m):
    return (x + m - 1) // m * m


@functools.lru_cache(maxsize=None)
def _make_gather(n_rows, w, b):
    b_per_w = b // NW
    n_chunks = b_per_w // CH
    mesh = plsc.VectorSubcoreMesh(core_axis_name="c", subcore_axis_name="s")

    @functools.partial(
        pl.kernel,
        mesh=mesh,
        compiler_params=pltpu.CompilerParams(needs_layout_passes=False),
        out_type=jax.ShapeDtypeStruct((b, w), jnp.float32),
        scratch_types=[
            pltpu.VMEM((CH,), jnp.int32),
            pltpu.VMEM((CH, w), jnp.float32),
            pltpu.SemaphoreType.DMA,
        ],
    )
    def gather_kernel(tab_hbm, idx_hbm, out_hbm, iv, rows, sem):
        cid = lax.axis_index("c")
        sid = lax.axis_index("s")
        wid = sid * NC + cid
        base = wid * b_per_w

        def body(j, carry):
            off = pl.multiple_of(base + j * CH, CH)
            pltpu.sync_copy(idx_hbm.at[pl.ds(off, CH)], iv)
            pltpu.async_copy(tab_hbm.at[iv], rows, sem).wait()
            pltpu.sync_copy(rows, out_hbm.at[pl.ds(off, CH)])
            return carry

        lax.fori_loop(0, n_chunks, body, 0)

    return gather_kernel


def _gather_rows(table, idx):
    n_rows, w = table.shape
    fn = _make_gather(n_rows, w, idx.shape[0])
    return fn(table, idx)


def _segsum_body(d_ref, m_ref, o_ref):
    k = pl.program_id(1)
    i = pl.program_id(0)
    rows = i * BN + lax.broadcasted_iota(jnp.int32, (BN, EC), 0)
    onehot_t = (d_ref[...] == rows).astype(jnp.float32)
    contrib = jnp.dot(onehot_t, m_ref[...],
                      preferred_element_type=jnp.float32)

    @pl.when(k == 0)
    def _():
        o_ref[...] = contrib

    @pl.when(k != 0)
    def _():
        o_ref[...] = o_ref[...] + contrib


def _segsum_tc(msgs, dst, n_pad):
    e_pad, w = msgs.shape
    grid = (n_pad // BN, e_pad // EC)
    return pl.pallas_call(
        _segsum_body,
        grid=grid,
        in_specs=[
            pl.BlockSpec((1, EC), lambda i, k: (0, k)),
            pl.BlockSpec((EC, w), lambda i, k: (k, 0)),
        ],
        out_specs=pl.BlockSpec((BN, w), lambda i, k: (i, 0)),
        out_shape=jax.ShapeDtypeStruct((n_pad, w), jnp.float32),
    )(dst.reshape(1, e_pad), msgs)


def _deg_body(s_ref, d_ref, o_ref):
    k = pl.program_id(1)
    i = pl.program_id(0)
    rows = i * BN + lax.broadcasted_iota(jnp.int32, (BN, EC), 0)
    so = jnp.sum((s_ref[...] == rows).astype(jnp.float32), axis=1,
                 keepdims=True)
    do = jnp.sum((d_ref[...] == rows).astype(jnp.float32), axis=1,
                 keepdims=True)
    contrib = jnp.concatenate([so, do], axis=1)

    @pl.when(k == 0)
    def _():
        o_ref[...] = contrib

    @pl.when(k != 0)
    def _():
        o_ref[...] = o_ref[...] + contrib


def _degrees(src, dst, n_pad):
    e_pad = src.shape[0]
    grid = (n_pad // BN, e_pad // EC)
    out = pl.pallas_call(
        _deg_body,
        grid=grid,
        in_specs=[
            pl.BlockSpec((1, EC), lambda i, k: (0, k)),
            pl.BlockSpec((1, EC), lambda i, k: (0, k)),
        ],
        out_specs=pl.BlockSpec((BN, 2), lambda i, k: (i, 0)),
        out_shape=jax.ShapeDtypeStruct((n_pad, 2), jnp.float32),
    )(src.reshape(1, e_pad), dst.reshape(1, e_pad))
    return out


def _mm_body(a_ref, b_ref, d_ref, o_ref, *, scale_min):
    acc = jnp.dot(a_ref[...], b_ref[...], preferred_element_type=jnp.float32)
    if scale_min is not None:
        j = pl.program_id(1)
        s = lax.rsqrt(jnp.maximum(d_ref[...], 1.0))
        acc = jnp.where(j >= scale_min, acc * s, acc)
    o_ref[...] = acc


def _fatmm(a, bmat, dego=None, bm=None, scale_min=None, bn=BN):
    n, k = a.shape
    m = bmat.shape[1]
    if bm is None:
        bm = m
    grid = (pl.cdiv(n, bn), m // bm)
    if dego is None:
        dego = jnp.zeros((n, 1), jnp.float32)
    return pl.pallas_call(
        functools.partial(_mm_body, scale_min=scale_min),
        grid=grid,
        in_specs=[
            pl.BlockSpec((bn, k), lambda i, j: (i, 0)),
            pl.BlockSpec((k, bm), lambda i, j: (0, j)),
            pl.BlockSpec((bn, 1), lambda i, j: (i, 0)),
        ],
        out_specs=pl.BlockSpec((bn, bm), lambda i, j: (i, j)),
        out_shape=jax.ShapeDtypeStruct((n, m), jnp.float32),
    )(a, bmat, dego)


def _combine_body(a_ref, p_ref, di_ref, do_ref, b_ref, o_ref, *,
                  use_do, use_bias, use_relu):
    s = lax.rsqrt(jnp.maximum(di_ref[...], 1.0))
    if use_do:
        s = s * lax.rsqrt(jnp.maximum(do_ref[...], 1.0))
    y = a_ref[...] + p_ref[...] * s
    if use_bias:
        y = y + b_ref[...]
    if use_relu:
        y = jnp.maximum(y, 0.0)
    o_ref[...] = y


def _combine(a, p, degi, dego, bias, relu, bn=BN):
    n, w = a.shape
    use_do = dego is not None
    use_bias = bias is not None
    if dego is None:
        dego = degi
    if bias is None:
        bias = jnp.zeros((1, w), jnp.float32)
    else:
        bias = bias.reshape(1, w)
    grid = (pl.cdiv(n, bn),)
    return pl.pallas_call(
        functools.partial(_combine_body, use_do=use_do, use_bias=use_bias,
                          use_relu=relu),
        grid=grid,
        in_specs=[
            pl.BlockSpec((bn, w), lambda i: (i, 0)),
            pl.BlockSpec((bn, w), lambda i: (i, 0)),
            pl.BlockSpec((bn, 1), lambda i: (i, 0)),
            pl.BlockSpec((bn, 1), lambda i: (i, 0)),
            pl.BlockSpec((1, w), lambda i: (0, 0)),
        ],
        out_specs=pl.BlockSpec((bn, w), lambda i: (i, 0)),
        out_shape=jax.ShapeDtypeStruct((n, w), jnp.float32),
    )(a, p, degi, dego, bias)


def _pad_idx(idx, e_pad, val):
    e = idx.shape[0]
    return jnp.concatenate(
        [idx, jnp.full((e_pad - e,), val, jnp.int32)])


def _conv(h, src_g, dst_m, n, n_pad, W, b, degi, dego):
    k = W.shape[1]
    m = W.shape[2]
    wcat = jnp.transpose(W, (1, 0, 2)).reshape(k, 3 * m)
    g = _fatmm(h, wcat, dego, bm=m, scale_min=1)
    g0 = g[:, :m]
    g1s = g[:, m:2 * m]
    g2s = g[:, 2 * m:]
    t = _segsum_tc(_gather_rows(g2s, src_g), dst_m, n_pad)
    u = _combine(g1s, t[:n], degi, dego, None, relu=False)
    q = _segsum_tc(_gather_rows(u, src_g), dst_m, n_pad)
    return _combine(g0, q[:n], degi, None, b, relu=True)


def _graph_embed(h, edge_index, n, W1, b1, W2, b2):
    n_pad = _round_up(n, BN)
    e = edge_index.shape[1]
    e_pad = _round_up(e, NW * CH)
    src = edge_index[0]
    dst = edge_index[1]
    src_g = _pad_idx(src, e_pad, 0)
    src_m = _pad_idx(src, e_pad, -1)
    dst_m = _pad_idx(dst, e_pad, -1)
    degs = _degrees(src_m, dst_m, n_pad)
    dego = degs[:n, 0:1]
    degi = degs[:n, 1:2]
    h = _conv(h, src_g, dst_m, n, n_pad, W1, b1, degi, dego)
    h = _conv(h, src_g, dst_m, n, n_pad, W2, b2, degi, dego)
    return h


def kernel(mm_edge_index, cc_edge_index, mc_edge_index, miRNA, circrna,
           samples, W_lin_m, W_lin_d, W_mm1, b_mm1, W_mm2, b_mm2,
           W_cc1, b_cc1, W_cc2, b_cc2, W_mc1, b_mc1, W_mc2, b_mc2):
    emb_mm_sim = _graph_embed(miRNA, mm_edge_index, MI_N,
                              W_mm1, b_mm1, W_mm2, b_mm2)
    emb_cc_sim = _graph_embed(circrna, cc_edge_index, CI_N,
                              W_cc1, b_cc1, W_cc2, b_cc2)
    h0 = jnp.concatenate([_fatmm(miRNA, W_lin_m), _fatmm(circrna, W_lin_d)],
                         axis=0)
    emb_ass = _graph_embed(h0, mc_edge_index, MC_N,
                           W_mc1, b_mc1, W_mc2, b_mc2)
    emb_mm = jnp.concatenate([emb_mm_sim, emb_ass[:MI_N]], axis=1)
    emb_cc = jnp.concatenate([emb_cc_sim, emb_ass[MI_N:]], axis=1)
    gmm = _gather_rows(emb_mm, samples[:, 0])
    gcc = _gather_rows(emb_cc, samples[:, 1])
    return jnp.concatenate([gmm, gcc], axis=1)


# speedup vs baseline: 1.0057x; 1.0057x over previous
import functools

import jax
import jax.numpy as jnp
from jax import lax
from jax.experimental import pallas as pl
from jax.experimental.pallas import tpu as pltpu
from jax.experimental.pallas import tpu_sc as plsc

MI_N = 2000
CI_N = 3000
MC_N = MI_N + CI_N

NC = 2
NS = 16
NW = NC * NS
CH = 128

BN = 256
EC = 256


def _round_up(x, m):
    return (x + m - 1) // m * m


@functools.lru_cache(maxsize=None)
def _make_gather(n_rows, w, b):
    b_per_w = b // NW
    n_chunks = b_per_w // CH
    mesh = plsc.VectorSubcoreMesh(core_axis_name="c", subcore_axis_name="s")

    @functools.partial(
        pl.kernel,
        mesh=mesh,
        compiler_params=pltpu.CompilerParams(needs_layout_passes=False),
        out_type=jax.ShapeDtypeStruct((b, w), jnp.float32),
        scratch_types=[
            pltpu.VMEM((CH,), jnp.int32),
            pltpu.VMEM((CH, w), jnp.float32),
            pltpu.SemaphoreType.DMA,
        ],
    )
    def gather_kernel(tab_hbm, idx_hbm, out_hbm, iv, rows, sem):
        cid = lax.axis_index("c")
        sid = lax.axis_index("s")
        wid = sid * NC + cid
        base = wid * b_per_w

        def body(j, carry):
            off = pl.multiple_of(base + j * CH, CH)
            pltpu.sync_copy(idx_hbm.at[pl.ds(off, CH)], iv)
            pltpu.async_copy(tab_hbm.at[iv], rows, sem).wait()
            pltpu.sync_copy(rows, out_hbm.at[pl.ds(off, CH)])
            return carry

        lax.fori_loop(0, n_chunks, body, 0)

    return gather_kernel


def _gather_rows(table, idx):
    n_rows, w = table.shape
    fn = _make_gather(n_rows, w, idx.shape[0])
    return fn(table, idx)


def _segsum_body(d_ref, m_ref, o_ref):
    k = pl.program_id(1)
    i = pl.program_id(0)
    rows = i * BN + lax.broadcasted_iota(jnp.int32, (BN, EC), 0)
    onehot_t = (d_ref[...] == rows).astype(jnp.bfloat16)
    contrib = jnp.dot(onehot_t, m_ref[...].astype(jnp.bfloat16),
                      preferred_element_type=jnp.float32)

    @pl.when(k == 0)
    def _():
        o_ref[...] = contrib

    @pl.when(k != 0)
    def _():
        o_ref[...] = o_ref[...] + contrib


def _segsum_tc(msgs, dst, n_pad):
    e_pad, w = msgs.shape
    grid = (n_pad // BN, e_pad // EC)
    return pl.pallas_call(
        _segsum_body,
        grid=grid,
        in_specs=[
            pl.BlockSpec((1, EC), lambda i, k: (0, k)),
            pl.BlockSpec((EC, w), lambda i, k: (k, 0)),
        ],
        out_specs=pl.BlockSpec((BN, w), lambda i, k: (i, 0)),
        out_shape=jax.ShapeDtypeStruct((n_pad, w), jnp.float32),
    )(dst.reshape(1, e_pad), msgs)


def _deg_body(s_ref, d_ref, o_ref):
    k = pl.program_id(1)
    i = pl.program_id(0)
    rows = i * BN + lax.broadcasted_iota(jnp.int32, (BN, EC), 0)
    so = jnp.sum((s_ref[...] == rows).astype(jnp.float32), axis=1,
                 keepdims=True)
    do = jnp.sum((d_ref[...] == rows).astype(jnp.float32), axis=1,
                 keepdims=True)
    contrib = jnp.concatenate([so, do], axis=1)

    @pl.when(k == 0)
    def _():
        o_ref[...] = contrib

    @pl.when(k != 0)
    def _():
        o_ref[...] = o_ref[...] + contrib


def _degrees(src, dst, n_pad):
    e_pad = src.shape[0]
    grid = (n_pad // BN, e_pad // EC)
    out = pl.pallas_call(
        _deg_body,
        grid=grid,
        in_specs=[
            pl.BlockSpec((1, EC), lambda i, k: (0, k)),
            pl.BlockSpec((1, EC), lambda i, k: (0, k)),
        ],
        out_specs=pl.BlockSpec((BN, 2), lambda i, k: (i, 0)),
        out_shape=jax.ShapeDtypeStruct((n_pad, 2), jnp.float32),
    )(src.reshape(1, e_pad), dst.reshape(1, e_pad))
    return out


def _mm_body(a_ref, b_ref, d_ref, o_ref, *, scale_min):
    acc = jnp.dot(a_ref[...], b_ref[...], preferred_element_type=jnp.float32)
    if scale_min is not None:
        j = pl.program_id(1)
        s = lax.rsqrt(jnp.maximum(d_ref[...], 1.0))
        acc = jnp.where(j >= scale_min, acc * s, acc)
    o_ref[...] = acc


def _fatmm(a, bmat, dego=None, bm=None, scale_min=None, bn=BN):
    n, k = a.shape
    m = bmat.shape[1]
    if bm is None:
        bm = m
    grid = (pl.cdiv(n, bn), m // bm)
    if dego is None:
        dego = jnp.zeros((n, 1), jnp.float32)
    return pl.pallas_call(
        functools.partial(_mm_body, scale_min=scale_min),
        grid=grid,
        in_specs=[
            pl.BlockSpec((bn, k), lambda i, j: (i, 0)),
            pl.BlockSpec((k, bm), lambda i, j: (0, j)),
            pl.BlockSpec((bn, 1), lambda i, j: (i, 0)),
        ],
        out_specs=pl.BlockSpec((bn, bm), lambda i, j: (i, j)),
        out_shape=jax.ShapeDtypeStruct((n, m), jnp.float32),
    )(a, bmat, dego)


def _combine_body(a_ref, p_ref, di_ref, do_ref, b_ref, o_ref, *,
                  use_do, use_bias, use_relu):
    s = lax.rsqrt(jnp.maximum(di_ref[...], 1.0))
    if use_do:
        s = s * lax.rsqrt(jnp.maximum(do_ref[...], 1.0))
    y = a_ref[...] + p_ref[...] * s
    if use_bias:
        y = y + b_ref[...]
    if use_relu:
        y = jnp.maximum(y, 0.0)
    o_ref[...] = y


def _combine(a, p, degi, dego, bias, relu, bn=BN):
    n, w = a.shape
    use_do = dego is not None
    use_bias = bias is not None
    if dego is None:
        dego = degi
    if bias is None:
        bias = jnp.zeros((1, w), jnp.float32)
    else:
        bias = bias.reshape(1, w)
    grid = (pl.cdiv(n, bn),)
    return pl.pallas_call(
        functools.partial(_combine_body, use_do=use_do, use_bias=use_bias,
                          use_relu=relu),
        grid=grid,
        in_specs=[
            pl.BlockSpec((bn, w), lambda i: (i, 0)),
            pl.BlockSpec((bn, w), lambda i: (i, 0)),
            pl.BlockSpec((bn, 1), lambda i: (i, 0)),
            pl.BlockSpec((bn, 1), lambda i: (i, 0)),
            pl.BlockSpec((1, w), lambda i: (0, 0)),
        ],
        out_specs=pl.BlockSpec((bn, w), lambda i: (i, 0)),
        out_shape=jax.ShapeDtypeStruct((n, w), jnp.float32),
    )(a, p, degi, dego, bias)


def _pad_idx(idx, e_pad, val):
    e = idx.shape[0]
    return jnp.concatenate(
        [idx, jnp.full((e_pad - e,), val, jnp.int32)])


def _conv(h, src_g, dst_m, n, n_pad, W, b, degi, dego):
    k = W.shape[1]
    m = W.shape[2]
    wcat = jnp.transpose(W, (1, 0, 2)).reshape(k, 3 * m)
    g = _fatmm(h, wcat, dego, bm=m, scale_min=1)
    g0 = g[:, :m]
    g1s = g[:, m:2 * m]
    g2s = g[:, 2 * m:]
    t = _segsum_tc(_gather_rows(g2s, src_g), dst_m, n_pad)
    u = _combine(g1s, t[:n], degi, dego, None, relu=False)
    q = _segsum_tc(_gather_rows(u, src_g), dst_m, n_pad)
    return _combine(g0, q[:n], degi, None, b, relu=True)


def _graph_embed(h, edge_index, n, W1, b1, W2, b2):
    n_pad = _round_up(n, BN)
    e = edge_index.shape[1]
    e_pad = _round_up(e, NW * CH)
    src = edge_index[0]
    dst = edge_index[1]
    src_g = _pad_idx(src, e_pad, 0)
    src_m = _pad_idx(src, e_pad, -1)
    dst_m = _pad_idx(dst, e_pad, -1)
    degs = _degrees(src_m, dst_m, n_pad)
    dego = degs[:n, 0:1]
    degi = degs[:n, 1:2]
    h = _conv(h, src_g, dst_m, n, n_pad, W1, b1, degi, dego)
    h = _conv(h, src_g, dst_m, n, n_pad, W2, b2, degi, dego)
    return h


def kernel(mm_edge_index, cc_edge_index, mc_edge_index, miRNA, circrna,
           samples, W_lin_m, W_lin_d, W_mm1, b_mm1, W_mm2, b_mm2,
           W_cc1, b_cc1, W_cc2, b_cc2, W_mc1, b_mc1, W_mc2, b_mc2):
    emb_mm_sim = _graph_embed(miRNA, mm_edge_index, MI_N,
                              W_mm1, b_mm1, W_mm2, b_mm2)
    emb_cc_sim = _graph_embed(circrna, cc_edge_index, CI_N,
                              W_cc1, b_cc1, W_cc2, b_cc2)
    h0 = jnp.concatenate([_fatmm(miRNA, W_lin_m), _fatmm(circrna, W_lin_d)],
                         axis=0)
    emb_ass = _graph_embed(h0, mc_edge_index, MC_N,
                           W_mc1, b_mc1, W_mc2, b_mc2)
    emb_mm = jnp.concatenate([emb_mm_sim, emb_ass[:MI_N]], axis=1)
    emb_cc = jnp.concatenate([emb_cc_sim, emb_ass[MI_N:]], axis=1)
    gmm = _gather_rows(emb_mm, samples[:, 0])
    gcc = _gather_rows(emb_cc, samples[:, 1])
    return jnp.concatenate([gmm, gcc], axis=1)


# speedup vs baseline: 6.4348x; 6.3983x over previous
import functools

import jax
import jax.numpy as jnp
from jax import lax
from jax.experimental import pallas as pl
from jax.experimental.pallas import tpu as pltpu
from jax.experimental.pallas import tpu_sc as plsc

MI_N = 2000
CI_N = 3000
MC_N = MI_N + CI_N

NC = 2
NS = 16
NW = NC * NS
CH = 128

BN = 256
EC = 256


def _round_up(x, m):
    return (x + m - 1) // m * m


@functools.lru_cache(maxsize=None)
def _make_gather(n_rows, w, b):
    b_per_w = b // NW
    n_chunks = b_per_w // CH
    mesh = plsc.VectorSubcoreMesh(core_axis_name="c", subcore_axis_name="s")

    @functools.partial(
        pl.kernel,
        mesh=mesh,
        compiler_params=pltpu.CompilerParams(needs_layout_passes=False),
        out_type=jax.ShapeDtypeStruct((b, w), jnp.float32),
        scratch_types=[
            pltpu.VMEM((CH,), jnp.int32),
            pltpu.VMEM((CH, w), jnp.float32),
            pltpu.SemaphoreType.DMA,
        ],
    )
    def gather_kernel(tab_hbm, idx_hbm, out_hbm, iv, rows, sem):
        cid = lax.axis_index("c")
        sid = lax.axis_index("s")
        wid = sid * NC + cid
        base = wid * b_per_w

        def body(j, carry):
            off = pl.multiple_of(base + j * CH, CH)
            pltpu.sync_copy(idx_hbm.at[pl.ds(off, CH)], iv)
            pltpu.async_copy(tab_hbm.at[iv], rows, sem).wait()
            pltpu.sync_copy(rows, out_hbm.at[pl.ds(off, CH)])
            return carry

        lax.fori_loop(0, n_chunks, body, 0)

    return gather_kernel


def _gather_rows(table, idx):
    n_rows, w = table.shape
    fn = _make_gather(n_rows, w, idx.shape[0])
    return fn(table, idx)


def _segsum_body(d_ref, m_ref, o_ref, *, n_pad):
    k = pl.program_id(0)
    rows = lax.broadcasted_iota(jnp.int32, (n_pad, EC), 0)
    onehot_t = (d_ref[...] == rows).astype(jnp.bfloat16)
    contrib = jnp.dot(onehot_t, m_ref[...].astype(jnp.bfloat16),
                      preferred_element_type=jnp.float32)

    @pl.when(k == 0)
    def _():
        o_ref[...] = contrib

    @pl.when(k != 0)
    def _():
        o_ref[...] = o_ref[...] + contrib


def _segsum_tc(msgs, dst, n_pad):
    e_pad, w = msgs.shape
    grid = (e_pad // EC,)
    return pl.pallas_call(
        functools.partial(_segsum_body, n_pad=n_pad),
        grid=grid,
        in_specs=[
            pl.BlockSpec((1, EC), lambda k: (0, k)),
            pl.BlockSpec((EC, w), lambda k: (k, 0)),
        ],
        out_specs=pl.BlockSpec((n_pad, w), lambda k: (0, 0)),
        out_shape=jax.ShapeDtypeStruct((n_pad, w), jnp.float32),
    )(dst.reshape(1, e_pad), msgs)


def _deg_body(s_ref, d_ref, o_ref, *, n_pad):
    k = pl.program_id(0)
    rows = lax.broadcasted_iota(jnp.int32, (n_pad, EC), 0)
    so = jnp.sum((s_ref[...] == rows).astype(jnp.float32), axis=1,
                 keepdims=True)
    do = jnp.sum((d_ref[...] == rows).astype(jnp.float32), axis=1,
                 keepdims=True)
    contrib = jnp.concatenate([so, do], axis=1)

    @pl.when(k == 0)
    def _():
        o_ref[...] = contrib

    @pl.when(k != 0)
    def _():
        o_ref[...] = o_ref[...] + contrib


def _degrees(src, dst, n_pad):
    e_pad = src.shape[0]
    grid = (e_pad // EC,)
    out = pl.pallas_call(
        functools.partial(_deg_body, n_pad=n_pad),
        grid=grid,
        in_specs=[
            pl.BlockSpec((1, EC), lambda k: (0, k)),
            pl.BlockSpec((1, EC), lambda k: (0, k)),
        ],
        out_specs=pl.BlockSpec((n_pad, 2), lambda k: (0, 0)),
        out_shape=jax.ShapeDtypeStruct((n_pad, 2), jnp.float32),
    )(src.reshape(1, e_pad), dst.reshape(1, e_pad))
    return out


def _mm_body(a_ref, b_ref, d_ref, o_ref, *, scale_min):
    acc = jnp.dot(a_ref[...], b_ref[...], preferred_element_type=jnp.float32)
    if scale_min is not None:
        j = pl.program_id(1)
        s = lax.rsqrt(jnp.maximum(d_ref[...], 1.0))
        acc = jnp.where(j >= scale_min, acc * s, acc)
    o_ref[...] = acc


def _fatmm(a, bmat, dego=None, bm=None, scale_min=None, bn=BN):
    n, k = a.shape
    m = bmat.shape[1]
    if bm is None:
        bm = m
    grid = (pl.cdiv(n, bn), m // bm)
    if dego is None:
        dego = jnp.zeros((n, 1), jnp.float32)
    return pl.pallas_call(
        functools.partial(_mm_body, scale_min=scale_min),
        grid=grid,
        in_specs=[
            pl.BlockSpec((bn, k), lambda i, j: (i, 0)),
            pl.BlockSpec((k, bm), lambda i, j: (0, j)),
            pl.BlockSpec((bn, 1), lambda i, j: (i, 0)),
        ],
        out_specs=pl.BlockSpec((bn, bm), lambda i, j: (i, j)),
        out_shape=jax.ShapeDtypeStruct((n, m), jnp.float32),
    )(a, bmat, dego)


def _combine_body(a_ref, p_ref, di_ref, do_ref, b_ref, o_ref, *,
                  use_do, use_bias, use_relu):
    s = lax.rsqrt(jnp.maximum(di_ref[...], 1.0))
    if use_do:
        s = s * lax.rsqrt(jnp.maximum(do_ref[...], 1.0))
    y = a_ref[...] + p_ref[...] * s
    if use_bias:
        y = y + b_ref[...]
    if use_relu:
        y = jnp.maximum(y, 0.0)
    o_ref[...] = y


def _combine(a, p, degi, dego, bias, relu, bn=BN):
    n, w = a.shape
    use_do = dego is not None
    use_bias = bias is not None
    if dego is None:
        dego = degi
    if bias is None:
        bias = jnp.zeros((1, w), jnp.float32)
    else:
        bias = bias.reshape(1, w)
    grid = (pl.cdiv(n, bn),)
    return pl.pallas_call(
        functools.partial(_combine_body, use_do=use_do, use_bias=use_bias,
                          use_relu=relu),
        grid=grid,
        in_specs=[
            pl.BlockSpec((bn, w), lambda i: (i, 0)),
            pl.BlockSpec((bn, w), lambda i: (i, 0)),
            pl.BlockSpec((bn, 1), lambda i: (i, 0)),
            pl.BlockSpec((bn, 1), lambda i: (i, 0)),
            pl.BlockSpec((1, w), lambda i: (0, 0)),
        ],
        out_specs=pl.BlockSpec((bn, w), lambda i: (i, 0)),
        out_shape=jax.ShapeDtypeStruct((n, w), jnp.float32),
    )(a, p, degi, dego, bias)


def _pad_idx(idx, e_pad, val):
    e = idx.shape[0]
    return jnp.concatenate(
        [idx, jnp.full((e_pad - e,), val, jnp.int32)])


def _conv(h, src_g, dst_m, n, n_pad, W, b, degi, dego):
    k = W.shape[1]
    m = W.shape[2]
    wcat = jnp.transpose(W, (1, 0, 2)).reshape(k, 3 * m)
    g = _fatmm(h, wcat, dego, bm=m, scale_min=1)
    g0 = g[:, :m]
    g1s = g[:, m:2 * m]
    g2s = g[:, 2 * m:]
    t = _segsum_tc(_gather_rows(g2s, src_g), dst_m, n_pad)
    u = _combine(g1s, t[:n], degi, dego, None, relu=False)
    q = _segsum_tc(_gather_rows(u, src_g), dst_m, n_pad)
    return _combine(g0, q[:n], degi, None, b, relu=True)


def _graph_embed(h, edge_index, n, W1, b1, W2, b2):
    n_pad = _round_up(n, BN)
    e = edge_index.shape[1]
    e_pad = _round_up(e, NW * CH)
    src = edge_index[0]
    dst = edge_index[1]
    src_g = _pad_idx(src, e_pad, 0)
    src_m = _pad_idx(src, e_pad, -1)
    dst_m = _pad_idx(dst, e_pad, -1)
    degs = _degrees(src_m, dst_m, n_pad)
    dego = degs[:n, 0:1]
    degi = degs[:n, 1:2]
    h = _conv(h, src_g, dst_m, n, n_pad, W1, b1, degi, dego)
    h = _conv(h, src_g, dst_m, n, n_pad, W2, b2, degi, dego)
    return h


def kernel(mm_edge_index, cc_edge_index, mc_edge_index, miRNA, circrna,
           samples, W_lin_m, W_lin_d, W_mm1, b_mm1, W_mm2, b_mm2,
           W_cc1, b_cc1, W_cc2, b_cc2, W_mc1, b_mc1, W_mc2, b_mc2):
    emb_mm_sim = _graph_embed(miRNA, mm_edge_index, MI_N,
                              W_mm1, b_mm1, W_mm2, b_mm2)
    emb_cc_sim = _graph_embed(circrna, cc_edge_index, CI_N,
                              W_cc1, b_cc1, W_cc2, b_cc2)
    h0 = jnp.concatenate([_fatmm(miRNA, W_lin_m), _fatmm(circrna, W_lin_d)],
                         axis=0)
    emb_ass = _graph_embed(h0, mc_edge_index, MC_N,
                           W_mc1, b_mc1, W_mc2, b_mc2)
    emb_mm = jnp.concatenate([emb_mm_sim, emb_ass[:MI_N]], axis=1)
    emb_cc = jnp.concatenate([emb_cc_sim, emb_ass[MI_N:]], axis=1)
    gmm = _gather_rows(emb_mm, samples[:, 0])
    gcc = _gather_rows(emb_cc, samples[:, 1])
    return jnp.concatenate([gmm, gcc], axis=1)
